# baseline (speedup 1.0000x reference)
import functools

import numpy as np
import jax
import jax.numpy as jnp
from jax.experimental import pallas as pl
from jax.experimental.pallas import tpu as pltpu
from jax.experimental.pallas import tpu_sc as plsc

_LANES = 16
_WIN = 128


@functools.partial(jax.jit, static_argnames=("num_idx", "emb_dim"))
def _sc_gather(x_flat, off_flat, w_flat, *, num_idx, emb_dim):
    mesh = plsc.VectorSubcoreMesh(core_axis_name="core",
                                  subcore_axis_name="subcore")

    @functools.partial(
        pl.kernel,
        out_type=jax.ShapeDtypeStruct((num_idx, emb_dim), jnp.float32),
        mesh=mesh,
        scratch_types=[pltpu.VMEM((1, _WIN), jnp.int32)],
        compiler_params=pltpu.CompilerParams(use_tc_tiling_on_sc=False),
    )
    def k(x_hbm, off_hbm, w_hbm, o_hbm, idx_scratch):
        def body(x_vmem, off_vmem, o_vmem):
            @pl.loop(0, _WIN, step=_LANES)
            def _(c):
                sl = pl.ds(c, _LANES)
                idx_scratch[0, sl] = x_vmem[0, sl] + off_vmem[0, sl]

            pltpu.sync_copy(w_hbm.at[idx_scratch.at[0]], o_vmem)

        pltpu.emit_pipeline(
            body,
            grid=(num_idx // _WIN,),
            in_specs=[
                pl.BlockSpec((1, _WIN), index_map=lambda i: (0, i)),
                pl.BlockSpec((1, _WIN), index_map=lambda i: (0, i)),
            ],
            out_specs=[pl.BlockSpec((_WIN, emb_dim), index_map=lambda i: (i, 0))],
            core_axis_name=("core", "subcore"),
            dimension_semantics=(pltpu.PARALLEL,),
        )(x_hbm, off_hbm, o_hbm)

    return k(x_flat, off_flat, w_flat)


def kernel(x_cat, W):
    B, F = x_cat.shape
    _, V1, ED = W.shape
    num_idx = B * F
    x_flat = x_cat.astype(jnp.int32).reshape(1, num_idx)
    off = np.tile(np.arange(F, dtype=np.int32) * V1, B).reshape(1, num_idx)
    w_flat = W.reshape(F * V1, ED)
    out = _sc_gather(x_flat, jnp.asarray(off), w_flat,
                     num_idx=num_idx, emb_dim=ED)
    return out.reshape(B, F * ED)


# speedup vs baseline: 1.8853x; 1.8853x over previous
import functools

import jax
import jax.numpy as jnp
from jax.experimental import pallas as pl
from jax.experimental.pallas import tpu as pltpu
from jax.experimental.pallas import tpu_sc as plsc

_NW = 32
_WIN = 128
_CHUNK = 512
_RPW = _CHUNK // _WIN


@functools.partial(jax.jit, static_argnames=("num_fields", "batch", "emb_dim"))
def _sc_embed(x_t, w, *, num_fields, batch, emb_dim):
    mesh = plsc.VectorSubcoreMesh(core_axis_name="core",
                                  subcore_axis_name="subcore")
    num_chunks = batch // _CHUNK
    num_units = num_fields * num_chunks
    units_per_worker = num_units // _NW

    @functools.partial(
        pl.kernel,
        out_type=jax.ShapeDtypeStruct((batch, num_fields, emb_dim),
                                      jnp.float32),
        mesh=mesh,
        scratch_types=[
            pltpu.VMEM((_RPW, _WIN), jnp.int32),
            pltpu.VMEM((_CHUNK, emb_dim), jnp.float32),
            pltpu.SemaphoreType.DMA,
        ],
        compiler_params=pltpu.CompilerParams(use_tc_tiling_on_sc=False),
    )
    def k(x_hbm, w_hbm, o_hbm, idx_v, rows_v, sem):
        wid = jax.lax.axis_index("subcore") * 2 + jax.lax.axis_index("core")

        @pl.loop(0, units_per_worker)
        def _(t):
            u = t * _NW + wid
            j = u % num_fields
            c = u // num_fields
            pltpu.sync_copy(x_hbm.at[j, pl.ds(c * _RPW, _RPW)], idx_v)
            for r in range(_RPW):
                pltpu.async_copy(
                    w_hbm.at[j].at[idx_v.at[r]],
                    rows_v.at[pl.ds(r * _WIN, _WIN)],
                    sem,
                )
            for r in range(_RPW):
                pltpu.make_async_copy(
                    w_hbm.at[j].at[idx_v.at[r]],
                    rows_v.at[pl.ds(r * _WIN, _WIN)],
                    sem,
                ).wait()
            pltpu.sync_copy(rows_v, o_hbm.at[pl.ds(c * _CHUNK, _CHUNK), j])

    return k(x_t, w)


def kernel(x_cat, W):
    B, F = x_cat.shape
    _, _, ED = W.shape
    x_t = jnp.transpose(x_cat).astype(jnp.int32).reshape(F, B // _WIN, _WIN)
    out = _sc_embed(x_t, W, num_fields=F, batch=B, emb_dim=ED)
    return out.reshape(B, F * ED)


# speedup vs baseline: 3.8000x; 2.0156x over previous
import dataclasses
import functools

import jax
import jax.numpy as jnp
from jax.experimental import pallas as pl
from jax.experimental.pallas import tpu as pltpu
from jax.experimental.pallas import tpu_sc as plsc

def _no_layout_passes(cp):
    if "needs_layout_passes" in pltpu.CompilerParams.__dataclass_fields__:
        cp = dataclasses.replace(cp, needs_layout_passes=False)
    return cp


_NW = 32
_WIN = 128
_CHUNK = 512
_RPW = _CHUNK // _WIN


def _sc_pack(w_t, w_tail, v_pad):
    F, ED, V = w_t.shape
    vmax = V - 1
    v_main = (vmax // _WIN) * _WIN
    n_main = v_main // _WIN
    total_main = F * n_main
    iters = (total_main + _NW - 1) // _NW
    mesh = plsc.VectorSubcoreMesh(core_axis_name="core",
                                  subcore_axis_name="subcore")

    @functools.partial(
        pl.kernel,
        out_type=jax.ShapeDtypeStruct((F, v_pad // 8, 8 * ED), jnp.float32),
        mesh=mesh,
        scratch_types=[
            pltpu.VMEM((ED, _WIN), jnp.float32),
            pltpu.VMEM((ED, _WIN), jnp.float32),
        ],
        compiler_params=_no_layout_passes(
            pltpu.CompilerParams(use_tc_tiling_on_sc=True)),
    )
    def k(wt_hbm, tail_hbm, o_hbm, ibuf, obuf):
        wid = jax.lax.axis_index("subcore") * 2 + jax.lax.axis_index("core")
        iota = jax.lax.iota(jnp.int32, 16)
        col_base = [jnp.full((16,), kk, jnp.int32) for kk in range(8)]
        col_dst = [kk * ED + iota for kk in range(8)]

        def transpose_cols(n_c8):
            @pl.loop(0, n_c8)
            def _(c8):
                vbase = jnp.full((16,), c8 * 8, jnp.int32)
                vrow = jnp.full((16,), c8, jnp.int32)
                for kk in range(8):
                    src_col = vbase + col_base[kk]
                    row = plsc.load_gather(ibuf, [iota, src_col])
                    plsc.store_scatter(obuf, [vrow, col_dst[kk]], row)

        @pl.loop(0, iters)
        def _(it):
            t = it * _NW + wid

            @pl.when(t < total_main)
            def _():
                j = t // n_main
                tc = t % n_main
                pltpu.sync_copy(wt_hbm.at[j, :, pl.ds(tc * _WIN, _WIN)], ibuf)
                transpose_cols(_WIN // 8)
                pltpu.sync_copy(obuf, o_hbm.at[j, pl.ds(tc * ED, ED)])

        @pl.when(wid < F)
        def _():
            j = wid
            pltpu.sync_copy(tail_hbm.at[j], ibuf.at[pl.ds(0, 8)])
            pltpu.sync_copy(ibuf.at[pl.ds(0, 8)],
                            o_hbm.at[j, pl.ds(v_main // 8, 8)])

    return k(w_t, w_tail)


def _sc_embed(x_t, w, *, num_fields, batch, emb_dim):
    mesh = plsc.VectorSubcoreMesh(core_axis_name="core",
                                  subcore_axis_name="subcore")
    num_chunks = batch // _CHUNK
    num_units = num_fields * num_chunks
    units_per_worker = num_units // _NW

    @functools.partial(
        pl.kernel,
        out_type=jax.ShapeDtypeStruct((batch, num_fields, emb_dim),
                                      jnp.float32),
        mesh=mesh,
        scratch_types=[
            pltpu.VMEM((_RPW, _WIN), jnp.int32),
            pltpu.VMEM((_CHUNK, emb_dim), jnp.float32),
            pltpu.SemaphoreType.DMA,
        ],
        compiler_params=pltpu.CompilerParams(use_tc_tiling_on_sc=False),
    )
    def k(x_hbm, w_hbm, o_hbm, idx_v, rows_v, sem):
        wid = jax.lax.axis_index("subcore") * 2 + jax.lax.axis_index("core")

        @pl.loop(0, units_per_worker)
        def _(t):
            u = t * _NW + wid
            j = u % num_fields
            c = u // num_fields
            pltpu.sync_copy(x_hbm.at[j, pl.ds(c * _RPW, _RPW)], idx_v)
            for r in range(_RPW):
                pltpu.async_copy(
                    w_hbm.at[j].at[idx_v.at[r]],
                    rows_v.at[pl.ds(r * _WIN, _WIN)],
                    sem,
                )
            for r in range(_RPW):
                pltpu.make_async_copy(
                    w_hbm.at[j].at[idx_v.at[r]],
                    rows_v.at[pl.ds(r * _WIN, _WIN)],
                    sem,
                ).wait()
            pltpu.sync_copy(rows_v, o_hbm.at[pl.ds(c * _CHUNK, _CHUNK), j])

    return k(x_t, w)


@functools.partial(jax.jit, static_argnames=("num_fields", "batch", "emb_dim"))
def _embed_pipeline(x_t, W, *, num_fields, batch, emb_dim):
    V1 = W.shape[1]
    v_pad = ((V1 + 63) // 64) * 64
    v_main = (((V1 - 1) // _WIN) * _WIN)
    tail = jnp.pad(W[:, v_main:, :], ((0, 0), (0, v_main + 64 - V1), (0, 0)))
    w_tail = tail.reshape(num_fields, 8, 8 * emb_dim)
    w_pack = _sc_pack(jnp.transpose(W, (0, 2, 1)), w_tail, v_pad)
    w_lin = w_pack.reshape(num_fields, v_pad, emb_dim)
    return _sc_embed(x_t, w_lin, num_fields=num_fields, batch=batch,
                     emb_dim=emb_dim)


def kernel(x_cat, W):
    B, F = x_cat.shape
    _, _, ED = W.shape
    x_t = jnp.transpose(x_cat).astype(jnp.int32).reshape(F, B // _WIN, _WIN)
    out = _embed_pipeline(x_t, W, num_fields=F, batch=B, emb_dim=ED)
    return out.reshape(B, F * ED)


# speedup vs baseline: 4.7456x; 1.2488x over previous
import dataclasses
import functools

import jax
import jax.numpy as jnp
from jax.experimental import pallas as pl
from jax.experimental.pallas import tpu as pltpu
from jax.experimental.pallas import tpu_sc as plsc

def _no_layout_passes(cp):
    if "needs_layout_passes" in pltpu.CompilerParams.__dataclass_fields__:
        cp = dataclasses.replace(cp, needs_layout_passes=False)
    return cp


_NW = 32
_WIN = 128
_CHUNK = 512
_RPW = _CHUNK // _WIN


def _sc_pack(w_t, w_tail, v_pad):
    F, ED, V = w_t.shape
    vmax = V - 1
    v_main = (vmax // _WIN) * _WIN
    n_main = v_main // _WIN
    tpt = 11
    tw = tpt * _WIN
    gpt = n_main // tpt
    total_main = F * gpt
    iters = (total_main + _NW - 1) // _NW
    mesh = plsc.VectorSubcoreMesh(core_axis_name="core",
                                  subcore_axis_name="subcore")

    @functools.partial(
        pl.kernel,
        out_type=jax.ShapeDtypeStruct((F, v_pad // 8, 8 * ED), jnp.float32),
        mesh=mesh,
        scratch_types=[
            pltpu.VMEM((ED, tw), jnp.float32),
            pltpu.VMEM((tw * ED // _WIN, _WIN), jnp.float32),
        ],
        compiler_params=_no_layout_passes(
            pltpu.CompilerParams(use_tc_tiling_on_sc=True)),
    )
    def k(wt_hbm, tail_hbm, o_hbm, ibuf, obuf):
        wid = jax.lax.axis_index("subcore") * 2 + jax.lax.axis_index("core")
        iota = jax.lax.iota(jnp.int32, 16)
        col_base = [jnp.full((16,), kk, jnp.int32) for kk in range(8)]
        col_dst = [kk * ED + iota for kk in range(8)]

        def transpose_cols(n_c8):
            @pl.loop(0, n_c8)
            def _(c8):
                vbase = jnp.full((16,), c8 * 8, jnp.int32)
                vrow = jnp.full((16,), c8, jnp.int32)
                for kk in range(8):
                    src_col = vbase + col_base[kk]
                    row = plsc.load_gather(ibuf, [iota, src_col])
                    plsc.store_scatter(obuf, [vrow, col_dst[kk]], row)

        @pl.loop(0, iters)
        def _(it):
            t = it * _NW + wid

            @pl.when(t < total_main)
            def _():
                j = t // gpt
                g = t % gpt
                pltpu.sync_copy(wt_hbm.at[j, :, pl.ds(g * tw, tw)], ibuf)
                transpose_cols(tw // 8)
                rows = tw * ED // _WIN
                pltpu.sync_copy(obuf, o_hbm.at[j, pl.ds(g * rows, rows)])

        @pl.when(wid < F)
        def _():
            j = wid
            pltpu.sync_copy(tail_hbm.at[j], obuf.at[pl.ds(0, 8)])
            pltpu.sync_copy(obuf.at[pl.ds(0, 8)],
                            o_hbm.at[j, pl.ds(v_main // 8, 8)])

    return k(w_t, w_tail)


def _sc_embed(x_t, w, *, num_fields, batch, emb_dim):
    mesh = plsc.VectorSubcoreMesh(core_axis_name="core",
                                  subcore_axis_name="subcore")
    num_chunks = batch // _CHUNK
    num_units = num_fields * num_chunks
    units_per_worker = num_units // _NW

    @functools.partial(
        pl.kernel,
        out_type=jax.ShapeDtypeStruct((batch, num_fields, emb_dim),
                                      jnp.float32),
        mesh=mesh,
        scratch_types=[
            pltpu.VMEM((_RPW, _WIN), jnp.int32),
            pltpu.VMEM((_CHUNK, emb_dim), jnp.float32),
            pltpu.SemaphoreType.DMA,
        ],
        compiler_params=pltpu.CompilerParams(use_tc_tiling_on_sc=False),
    )
    def k(x_hbm, w_hbm, o_hbm, idx_v, rows_v, sem):
        wid = jax.lax.axis_index("subcore") * 2 + jax.lax.axis_index("core")

        @pl.loop(0, units_per_worker)
        def _(t):
            u = t * _NW + wid
            j = u % num_fields
            c = u // num_fields
            pltpu.sync_copy(x_hbm.at[j, pl.ds(c * _RPW, _RPW)], idx_v)
            for r in range(_RPW):
                pltpu.async_copy(
                    w_hbm.at[j].at[idx_v.at[r]],
                    rows_v.at[pl.ds(r * _WIN, _WIN)],
                    sem,
                )
            for r in range(_RPW):
                pltpu.make_async_copy(
                    w_hbm.at[j].at[idx_v.at[r]],
                    rows_v.at[pl.ds(r * _WIN, _WIN)],
                    sem,
                ).wait()
            pltpu.sync_copy(rows_v, o_hbm.at[pl.ds(c * _CHUNK, _CHUNK), j])

    return k(x_t, w)


@functools.partial(jax.jit, static_argnames=("num_fields", "batch", "emb_dim"))
def _embed_pipeline(x_t, W, *, num_fields, batch, emb_dim):
    V1 = W.shape[1]
    v_pad = ((V1 + 63) // 64) * 64
    v_main = (((V1 - 1) // _WIN) * _WIN)
    tail = jnp.pad(W[:, v_main:, :], ((0, 0), (0, v_main + 64 - V1), (0, 0)))
    w_tail = tail.reshape(num_fields, 8, 8 * emb_dim)
    w_pack = _sc_pack(jnp.transpose(W, (0, 2, 1)), w_tail, v_pad)
    w_lin = w_pack.reshape(num_fields, v_pad, emb_dim)
    return _sc_embed(x_t, w_lin, num_fields=num_fields, batch=batch,
                     emb_dim=emb_dim)


def kernel(x_cat, W):
    B, F = x_cat.shape
    _, _, ED = W.shape
    x_t = jnp.transpose(x_cat).astype(jnp.int32).reshape(F, B // _WIN, _WIN)
    out = _embed_pipeline(x_t, W, num_fields=F, batch=B, emb_dim=ED)
    return out.reshape(B, F * ED)


# speedup vs baseline: 6.6193x; 1.3948x over previous
import dataclasses
import functools

import jax
import jax.numpy as jnp
from jax.experimental import pallas as pl
from jax.experimental.pallas import tpu as pltpu
from jax.experimental.pallas import tpu_sc as plsc

def _no_layout_passes(cp):
    if "needs_layout_passes" in pltpu.CompilerParams.__dataclass_fields__:
        cp = dataclasses.replace(cp, needs_layout_passes=False)
    return cp


_NW = 32
_WIN = 128
_CHUNK = 512
_RPW = _CHUNK // _WIN


def _sc_pack(w_t, w_tail, v_pad):
    F, ED, V = w_t.shape
    vmax = V - 1
    v_main = (vmax // _WIN) * _WIN
    n_main = v_main // _WIN
    tpt = 11
    tw = tpt * _WIN
    gpt = n_main // tpt
    total_main = F * gpt
    iters = (total_main + _NW - 1) // _NW
    mesh = plsc.VectorSubcoreMesh(core_axis_name="core",
                                  subcore_axis_name="subcore")

    @functools.partial(
        pl.kernel,
        out_type=jax.ShapeDtypeStruct((F, v_pad // 8, 8 * ED), jnp.float32),
        mesh=mesh,
        scratch_types=[
            pltpu.VMEM((ED, tw), jnp.float32),
            pltpu.VMEM((tw * ED // _WIN, _WIN), jnp.float32),
        ],
        compiler_params=_no_layout_passes(
            pltpu.CompilerParams(use_tc_tiling_on_sc=True)),
    )
    def k(wt_hbm, tail_hbm, o_hbm, ibuf, obuf):
        wid = jax.lax.axis_index("subcore") * 2 + jax.lax.axis_index("core")
        iota = jax.lax.iota(jnp.int32, 16)
        col_base = [jnp.full((16,), kk, jnp.int32) for kk in range(8)]
        col_dst = [kk * ED + iota for kk in range(8)]

        def transpose_cols(n_c8):
            @pl.loop(0, n_c8 // 2)
            def _(c16):
                rows, metas = [], []
                for half in range(2):
                    c8 = c16 * 2 + half
                    vbase = jnp.full((16,), c8 * 8, jnp.int32)
                    vrow = jnp.full((16,), c8, jnp.int32)
                    for kk in range(8):
                        rows.append(
                            plsc.load_gather(ibuf, [iota, vbase + col_base[kk]]))
                        metas.append((vrow, col_dst[kk]))
                for row, (vrow, cd) in zip(rows, metas):
                    plsc.store_scatter(obuf, [vrow, cd], row)

        @pl.loop(0, iters)
        def _(it):
            t = it * _NW + wid

            @pl.when(t < total_main)
            def _():
                j = t // gpt
                g = t % gpt
                pltpu.sync_copy(wt_hbm.at[j, :, pl.ds(g * tw, tw)], ibuf)
                transpose_cols(tw // 8)
                rows = tw * ED // _WIN
                pltpu.sync_copy(obuf, o_hbm.at[j, pl.ds(g * rows, rows)])

        @pl.when(wid < F)
        def _():
            j = wid
            pltpu.sync_copy(tail_hbm.at[j], obuf.at[pl.ds(0, 8)])
            pltpu.sync_copy(obuf.at[pl.ds(0, 8)],
                            o_hbm.at[j, pl.ds(v_main // 8, 8)])

    return k(w_t, w_tail)


def _sc_embed(x_t, w, *, num_fields, batch, emb_dim):
    mesh = plsc.VectorSubcoreMesh(core_axis_name="core",
                                  subcore_axis_name="subcore")
    num_chunks = batch // _CHUNK
    num_units = num_fields * num_chunks
    units_per_worker = num_units // _NW

    @functools.partial(
        pl.kernel,
        out_type=jax.ShapeDtypeStruct((batch, num_fields, emb_dim),
                                      jnp.float32),
        mesh=mesh,
        scratch_types=[
            pltpu.VMEM((_RPW, _WIN), jnp.int32),
            pltpu.VMEM((_CHUNK, emb_dim), jnp.float32),
            pltpu.SemaphoreType.DMA,
        ],
        compiler_params=pltpu.CompilerParams(use_tc_tiling_on_sc=False),
    )
    def k(x_hbm, w_hbm, o_hbm, idx_v, rows_v, sem):
        wid = jax.lax.axis_index("subcore") * 2 + jax.lax.axis_index("core")

        @pl.loop(0, units_per_worker)
        def _(t):
            u = t * _NW + wid
            j = u % num_fields
            c = u // num_fields
            pltpu.sync_copy(x_hbm.at[j, pl.ds(c * _RPW, _RPW)], idx_v)
            for r in range(_RPW):
                pltpu.async_copy(
                    w_hbm.at[j].at[idx_v.at[r]],
                    rows_v.at[pl.ds(r * _WIN, _WIN)],
                    sem,
                )
            for r in range(_RPW):
                pltpu.make_async_copy(
                    w_hbm.at[j].at[idx_v.at[r]],
                    rows_v.at[pl.ds(r * _WIN, _WIN)],
                    sem,
                ).wait()
            pltpu.sync_copy(rows_v, o_hbm.at[pl.ds(c * _CHUNK, _CHUNK), j])

    return k(x_t, w)


@functools.partial(jax.jit, static_argnames=("num_fields", "batch", "emb_dim"))
def _embed_pipeline(x_t, W, *, num_fields, batch, emb_dim):
    V1 = W.shape[1]
    v_pad = ((V1 + 63) // 64) * 64
    v_main = (((V1 - 1) // _WIN) * _WIN)
    tail = jnp.pad(W[:, v_main:, :], ((0, 0), (0, v_main + 64 - V1), (0, 0)))
    w_tail = tail.reshape(num_fields, 8, 8 * emb_dim)
    w_pack = _sc_pack(jnp.transpose(W, (0, 2, 1)), w_tail, v_pad)
    w_lin = w_pack.reshape(num_fields, v_pad, emb_dim)
    return _sc_embed(x_t, w_lin, num_fields=num_fields, batch=batch,
                     emb_dim=emb_dim)


def kernel(x_cat, W):
    B, F = x_cat.shape
    _, _, ED = W.shape
    x_t = jnp.transpose(x_cat).astype(jnp.int32).reshape(F, B // _WIN, _WIN)
    out = _embed_pipeline(x_t, W, num_fields=F, batch=B, emb_dim=ED)
    return out.reshape(B, F * ED)


# speedup vs baseline: 7.6202x; 1.1512x over previous
import dataclasses
import functools

import jax
import jax.numpy as jnp
from jax.experimental import pallas as pl
from jax.experimental.pallas import tpu as pltpu
from jax.experimental.pallas import tpu_sc as plsc

def _no_layout_passes(cp):
    if "needs_layout_passes" in pltpu.CompilerParams.__dataclass_fields__:
        cp = dataclasses.replace(cp, needs_layout_passes=False)
    return cp


_NW = 32
_WIN = 128
_CHUNK = 512
_RPW = _CHUNK // _WIN


def _sc_pack(w_t, w_tail, v_pad):
    F, ED, V = w_t.shape
    vmax = V - 1
    v_main = (vmax // _WIN) * _WIN
    n_main = v_main // _WIN
    tpt = 11
    tw = tpt * _WIN
    gpt = n_main // tpt
    total_main = F * gpt
    iters = (total_main + _NW - 1) // _NW
    mesh = plsc.VectorSubcoreMesh(core_axis_name="core",
                                  subcore_axis_name="subcore")

    @functools.partial(
        pl.kernel,
        out_type=jax.ShapeDtypeStruct((F, v_pad // 8, 8 * ED), jnp.float32),
        mesh=mesh,
        scratch_types=[
            pltpu.VMEM((2, ED, tw), jnp.float32),
            pltpu.VMEM((2, tw * ED // _WIN, _WIN), jnp.float32),
            pltpu.SemaphoreType.DMA,
            pltpu.SemaphoreType.DMA,
            pltpu.SemaphoreType.DMA,
            pltpu.SemaphoreType.DMA,
        ],
        compiler_params=_no_layout_passes(
            pltpu.CompilerParams(use_tc_tiling_on_sc=True)),
    )
    def k(wt_hbm, tail_hbm, o_hbm, ibufs, obufs, si0, si1, so0, so1):
        wid = jax.lax.axis_index("subcore") * 2 + jax.lax.axis_index("core")
        iota = jax.lax.iota(jnp.int32, 16)
        col_base = [jnp.full((16,), kk, jnp.int32) for kk in range(8)]
        col_dst = [kk * ED + iota for kk in range(8)]
        orows = tw * ED // _WIN
        sin = (si0, si1)
        sout = (so0, so1)

        def coords(t):
            t = jnp.minimum(t, total_main - 1)
            return t // gpt, t % gpt

        def in_copy(k_task, b):
            j, g = coords(k_task * _NW + wid)
            return pltpu.make_async_copy(
                wt_hbm.at[j, :, pl.ds(g * tw, tw)], ibufs.at[b], sin[b])

        def out_copy(k_task, b):
            j, g = coords(k_task * _NW + wid)
            return pltpu.make_async_copy(
                obufs.at[b], o_hbm.at[j, pl.ds(g * orows, orows)], sout[b])

        def transpose_cols(ibuf, obuf, n_c8):
            @pl.loop(0, n_c8 // 2)
            def _(c16):
                rows, metas = [], []
                for half in range(2):
                    c8 = c16 * 2 + half
                    vbase = jnp.full((16,), c8 * 8, jnp.int32)
                    vrow = jnp.full((16,), c8, jnp.int32)
                    for kk in range(8):
                        rows.append(
                            plsc.load_gather(ibuf, [iota, vbase + col_base[kk]]))
                        metas.append((vrow, col_dst[kk]))
                for row, (vrow, cd) in zip(rows, metas):
                    plsc.store_scatter(obuf, [vrow, cd], row)

        in_copy(0, 0).start()
        in_copy(1, 1).start()

        @pl.loop(0, (iters + 1) // 2)
        def _(m):
            for half in range(2):
                kt = 2 * m + half
                in_copy(kt, half).wait()

                @pl.when(m > 0)
                def _():
                    out_copy(kt - 2, half).wait()

                transpose_cols(ibufs.at[half], obufs.at[half], tw // 8)
                out_copy(kt, half).start()
                in_copy(kt + 2, half).start()

        for half in range(2):
            kt = 2 * ((iters + 1) // 2) + half
            in_copy(kt, half).wait()
            out_copy(kt - 2, half).wait()

        @pl.when(wid < F)
        def _():
            j = wid
            pltpu.sync_copy(tail_hbm.at[j], obufs.at[0, pl.ds(0, 8)])
            pltpu.sync_copy(obufs.at[0, pl.ds(0, 8)],
                            o_hbm.at[j, pl.ds(v_main // 8, 8)])

    return k(w_t, w_tail)


def _sc_embed(x_t, w, *, num_fields, batch, emb_dim):
    mesh = plsc.VectorSubcoreMesh(core_axis_name="core",
                                  subcore_axis_name="subcore")
    num_chunks = batch // _CHUNK
    num_units = num_fields * num_chunks
    units_per_worker = num_units // _NW

    @functools.partial(
        pl.kernel,
        out_type=jax.ShapeDtypeStruct((batch, num_fields, emb_dim),
                                      jnp.float32),
        mesh=mesh,
        scratch_types=[
            pltpu.VMEM((_RPW, _WIN), jnp.int32),
            pltpu.VMEM((_CHUNK, emb_dim), jnp.float32),
            pltpu.SemaphoreType.DMA,
        ],
        compiler_params=pltpu.CompilerParams(use_tc_tiling_on_sc=False),
    )
    def k(x_hbm, w_hbm, o_hbm, idx_v, rows_v, sem):
        wid = jax.lax.axis_index("subcore") * 2 + jax.lax.axis_index("core")

        @pl.loop(0, units_per_worker)
        def _(t):
            u = t * _NW + wid
            j = u % num_fields
            c = u // num_fields
            pltpu.sync_copy(x_hbm.at[j, pl.ds(c * _RPW, _RPW)], idx_v)
            for r in range(_RPW):
                pltpu.async_copy(
                    w_hbm.at[j].at[idx_v.at[r]],
                    rows_v.at[pl.ds(r * _WIN, _WIN)],
                    sem,
                )
            for r in range(_RPW):
                pltpu.make_async_copy(
                    w_hbm.at[j].at[idx_v.at[r]],
                    rows_v.at[pl.ds(r * _WIN, _WIN)],
                    sem,
                ).wait()
            pltpu.sync_copy(rows_v, o_hbm.at[pl.ds(c * _CHUNK, _CHUNK), j])

    return k(x_t, w)


@functools.partial(jax.jit, static_argnames=("num_fields", "batch", "emb_dim"))
def _embed_pipeline(x_t, W, *, num_fields, batch, emb_dim):
    V1 = W.shape[1]
    v_pad = ((V1 + 63) // 64) * 64
    v_main = (((V1 - 1) // _WIN) * _WIN)
    tail = jnp.pad(W[:, v_main:, :], ((0, 0), (0, v_main + 64 - V1), (0, 0)))
    w_tail = tail.reshape(num_fields, 8, 8 * emb_dim)
    w_pack = _sc_pack(jnp.transpose(W, (0, 2, 1)), w_tail, v_pad)
    w_lin = w_pack.reshape(num_fields, v_pad, emb_dim)
    return _sc_embed(x_t, w_lin, num_fields=num_fields, batch=batch,
                     emb_dim=emb_dim)


def kernel(x_cat, W):
    B, F = x_cat.shape
    _, _, ED = W.shape
    x_t = jnp.transpose(x_cat).astype(jnp.int32).reshape(F, B // _WIN, _WIN)
    out = _embed_pipeline(x_t, W, num_fields=F, batch=B, emb_dim=ED)
    return out.reshape(B, F * ED)


# speedup vs baseline: 10.1955x; 1.3380x over previous
import dataclasses
import functools

import jax
import jax.numpy as jnp
from jax.experimental import pallas as pl
from jax.experimental.pallas import tpu as pltpu
from jax.experimental.pallas import tpu_sc as plsc

def _no_layout_passes(cp):
    if "needs_layout_passes" in pltpu.CompilerParams.__dataclass_fields__:
        cp = dataclasses.replace(cp, needs_layout_passes=False)
    return cp


_NW = 32
_WIN = 128
_CHUNK = 512
_RPW = _CHUNK // _WIN


def _sc_pack(w_t, w_tail, v_pad):
    F, ED, V = w_t.shape
    vmax = V - 1
    v_main = (vmax // _WIN) * _WIN
    n_main = v_main // _WIN
    tpt = 11
    tw = tpt * _WIN
    gpt = n_main // tpt
    total_main = F * gpt
    iters = (total_main + _NW - 1) // _NW
    mesh = plsc.VectorSubcoreMesh(core_axis_name="core",
                                  subcore_axis_name="subcore")

    @functools.partial(
        pl.kernel,
        out_type=jax.ShapeDtypeStruct((F, v_pad // 8, 8 * ED), jnp.float32),
        mesh=mesh,
        scratch_types=[
            pltpu.VMEM((2, ED, tw), jnp.float32),
            pltpu.VMEM((2, tw * ED // _WIN, _WIN), jnp.float32),
            pltpu.SemaphoreType.DMA,
            pltpu.SemaphoreType.DMA,
            pltpu.SemaphoreType.DMA,
            pltpu.SemaphoreType.DMA,
        ],
        compiler_params=_no_layout_passes(
            pltpu.CompilerParams(use_tc_tiling_on_sc=True)),
    )
    def k(wt_hbm, tail_hbm, o_hbm, ibufs, obufs, si0, si1, so0, so1):
        wid = jax.lax.axis_index("subcore") * 2 + jax.lax.axis_index("core")
        iota = jax.lax.iota(jnp.int32, 16)
        col_base = [jnp.full((16,), kk, jnp.int32) for kk in range(8)]
        col_dst = [kk * ED + iota for kk in range(8)]
        orows = tw * ED // _WIN
        sin = (si0, si1)
        sout = (so0, so1)

        def coords(t):
            t = jnp.minimum(t, total_main - 1)
            return t // gpt, t % gpt

        def in_copy(k_task, b):
            j, g = coords(k_task * _NW + wid)
            return pltpu.make_async_copy(
                wt_hbm.at[j, :, pl.ds(g * tw, tw)], ibufs.at[b], sin[b])

        def out_copy(k_task, b):
            j, g = coords(k_task * _NW + wid)
            return pltpu.make_async_copy(
                obufs.at[b], o_hbm.at[j, pl.ds(g * orows, orows)], sout[b])

        def transpose_cols(ibuf, obuf, n_c8):
            @pl.loop(0, n_c8 // 2)
            def _(c16):
                rows, metas = [], []
                for half in range(2):
                    c8 = c16 * 2 + half
                    vbase = jnp.full((16,), c8 * 8, jnp.int32)
                    vrow = jnp.full((16,), c8, jnp.int32)
                    for kk in range(8):
                        rows.append(
                            plsc.load_gather(ibuf, [iota, vbase + col_base[kk]]))
                        metas.append((vrow, col_dst[kk]))
                for row, (vrow, cd) in zip(rows, metas):
                    plsc.store_scatter(obuf, [vrow, cd], row)

        in_copy(0, 0).start()
        in_copy(1, 1).start()

        @pl.loop(0, (iters + 1) // 2)
        def _(m):
            for half in range(2):
                kt = 2 * m + half
                in_copy(kt, half).wait()

                @pl.when(m > 0)
                def _():
                    out_copy(kt - 2, half).wait()

                transpose_cols(ibufs.at[half], obufs.at[half], tw // 8)
                out_copy(kt, half).start()
                in_copy(kt + 2, half).start()

        for half in range(2):
            kt = 2 * ((iters + 1) // 2) + half
            in_copy(kt, half).wait()
            out_copy(kt - 2, half).wait()

        @pl.when(wid < F)
        def _():
            j = wid
            pltpu.sync_copy(tail_hbm.at[j], obufs.at[0, pl.ds(0, 8)])
            pltpu.sync_copy(obufs.at[0, pl.ds(0, 8)],
                            o_hbm.at[j, pl.ds(v_main // 8, 8)])

    return k(w_t, w_tail)


def _sc_embed(x_t, w, *, num_fields, batch, emb_dim):
    mesh = plsc.VectorSubcoreMesh(core_axis_name="core",
                                  subcore_axis_name="subcore")
    num_chunks = batch // _CHUNK
    num_units = num_fields * num_chunks
    units_per_worker = num_units // _NW

    @functools.partial(
        pl.kernel,
        out_type=jax.ShapeDtypeStruct((num_fields * emb_dim // 8,
                                       batch // _WIN, 8, _WIN), jnp.float32),
        mesh=mesh,
        scratch_types=[
            pltpu.VMEM((_RPW, _WIN), jnp.int32),
            pltpu.VMEM((_CHUNK, emb_dim), jnp.float32),
            pltpu.VMEM((2, 4, 8, _WIN), jnp.float32),
            pltpu.SemaphoreType.DMA,
        ],
        compiler_params=_no_layout_passes(
            pltpu.CompilerParams(use_tc_tiling_on_sc=False)),
    )
    def k(x_hbm, w_hbm, o_hbm, idx_v, rows_v, tbuf, sem):
        wid = jax.lax.axis_index("subcore") * 2 + jax.lax.axis_index("core")
        iota = jax.lax.iota(jnp.int32, 16)
        half_c = [jnp.full((16,), fp // 8, jnp.int32) for fp in range(16)]
        sub_c = [jnp.full((16,), fp % 8, jnp.int32) for fp in range(16)]
        feat_c = [jnp.full((16,), fp, jnp.int32) for fp in range(16)]

        @pl.loop(0, units_per_worker)
        def _(t):
            u = t * _NW + wid
            j = u % num_fields
            c = u // num_fields
            pltpu.sync_copy(x_hbm.at[j, pl.ds(c * _RPW, _RPW)], idx_v)
            for r in range(_RPW):
                pltpu.async_copy(
                    w_hbm.at[j].at[idx_v.at[r]],
                    rows_v.at[pl.ds(r * _WIN, _WIN)],
                    sem,
                )
            for r in range(_RPW):
                pltpu.make_async_copy(
                    w_hbm.at[j].at[idx_v.at[r]],
                    rows_v.at[pl.ds(r * _WIN, _WIN)],
                    sem,
                ).wait()

            @pl.loop(0, _CHUNK // 16)
            def _(m):
                vb = jnp.full((16,), m * 16, jnp.int32) + iota
                vtc = jnp.full((16,), m // 8, jnp.int32)
                vcc = jnp.full((16,), (m % 8) * 16, jnp.int32) + iota
                vecs = [plsc.load_gather(rows_v, [vb, feat_c[fp]])
                        for fp in range(16)]
                for fp in range(16):
                    plsc.store_scatter(
                        tbuf, [half_c[fp], vtc, sub_c[fp], vcc], vecs[fp])

            pltpu.sync_copy(tbuf, o_hbm.at[pl.ds(2 * j, 2), pl.ds(4 * c, 4)])

    return k(x_t, w)


@functools.partial(jax.jit, static_argnames=("num_fields", "batch", "emb_dim"))
def _embed_pipeline(x_t, W, *, num_fields, batch, emb_dim):
    V1 = W.shape[1]
    v_pad = ((V1 + 63) // 64) * 64
    v_main = (((V1 - 1) // _WIN) * _WIN)
    tail = jnp.pad(W[:, v_main:, :], ((0, 0), (0, v_main + 64 - V1), (0, 0)))
    w_tail = tail.reshape(num_fields, 8, 8 * emb_dim)
    w_pack = _sc_pack(jnp.transpose(W, (0, 2, 1)), w_tail, v_pad)
    w_lin = w_pack.reshape(num_fields, v_pad, emb_dim)
    return _sc_embed(x_t, w_lin, num_fields=num_fields, batch=batch,
                     emb_dim=emb_dim)


def kernel(x_cat, W):
    B, F = x_cat.shape
    _, _, ED = W.shape
    x_t = jnp.transpose(x_cat).astype(jnp.int32).reshape(F, B // _WIN, _WIN)
    out = _embed_pipeline(x_t, W, num_fields=F, batch=B, emb_dim=ED)
    return out.transpose(1, 3, 0, 2).reshape(B, F * ED)


# speedup vs baseline: 10.2139x; 1.0018x over previous
import dataclasses
import functools

import jax
import jax.numpy as jnp
from jax.experimental import pallas as pl
from jax.experimental.pallas import tpu as pltpu
from jax.experimental.pallas import tpu_sc as plsc

def _no_layout_passes(cp):
    if "needs_layout_passes" in pltpu.CompilerParams.__dataclass_fields__:
        cp = dataclasses.replace(cp, needs_layout_passes=False)
    return cp


_NW = 32
_WIN = 128
_CHUNK = 512
_RPW = _CHUNK // _WIN


def _sc_pack(w_t, w_tail, v_pad):
    F, ED, V = w_t.shape
    vmax = V - 1
    v_main = (vmax // _WIN) * _WIN
    n_main = v_main // _WIN
    tpt = 11
    tw = tpt * _WIN
    gpt = n_main // tpt
    total_main = F * gpt
    iters = (total_main + _NW - 1) // _NW
    mesh = plsc.VectorSubcoreMesh(core_axis_name="core",
                                  subcore_axis_name="subcore")

    @functools.partial(
        pl.kernel,
        out_type=jax.ShapeDtypeStruct((F, v_pad // 8, 8 * ED), jnp.float32),
        mesh=mesh,
        scratch_types=[
            pltpu.VMEM((2, ED, tw), jnp.float32),
            pltpu.VMEM((2, tw * ED // _WIN, _WIN), jnp.float32),
            pltpu.SemaphoreType.DMA,
            pltpu.SemaphoreType.DMA,
            pltpu.SemaphoreType.DMA,
            pltpu.SemaphoreType.DMA,
        ],
        compiler_params=_no_layout_passes(
            pltpu.CompilerParams(use_tc_tiling_on_sc=True)),
    )
    def k(wt_hbm, tail_hbm, o_hbm, ibufs, obufs, si0, si1, so0, so1):
        wid = jax.lax.axis_index("subcore") * 2 + jax.lax.axis_index("core")
        iota = jax.lax.iota(jnp.int32, 16)
        col_base = [jnp.full((16,), kk, jnp.int32) for kk in range(8)]
        col_dst = [kk * ED + iota for kk in range(8)]
        orows = tw * ED // _WIN
        sin = (si0, si1)
        sout = (so0, so1)

        def coords(t):
            t = jnp.minimum(t, total_main - 1)
            return t // gpt, t % gpt

        def in_copy(k_task, b):
            j, g = coords(k_task * _NW + wid)
            return pltpu.make_async_copy(
                wt_hbm.at[j, :, pl.ds(g * tw, tw)], ibufs.at[b], sin[b])

        def out_copy(k_task, b):
            j, g = coords(k_task * _NW + wid)
            return pltpu.make_async_copy(
                obufs.at[b], o_hbm.at[j, pl.ds(g * orows, orows)], sout[b])

        def transpose_cols(ibuf, obuf, n_c8):
            @pl.loop(0, n_c8 // 4)
            def _(c32):
                for pair in range(2):
                    rows, metas = [], []
                    for half in range(2):
                        c8 = c32 * 4 + pair * 2 + half
                        vbase = jnp.full((16,), c8 * 8, jnp.int32)
                        vrow = jnp.full((16,), c8, jnp.int32)
                        for kk in range(8):
                            rows.append(plsc.load_gather(
                                ibuf, [iota, vbase + col_base[kk]]))
                            metas.append((vrow, col_dst[kk]))
                    for row, (vrow, cd) in zip(rows, metas):
                        plsc.store_scatter(obuf, [vrow, cd], row)

        in_copy(0, 0).start()
        in_copy(1, 1).start()

        @pl.loop(0, (iters + 1) // 2)
        def _(m):
            for half in range(2):
                kt = 2 * m + half
                in_copy(kt, half).wait()

                @pl.when(m > 0)
                def _():
                    out_copy(kt - 2, half).wait()

                transpose_cols(ibufs.at[half], obufs.at[half], tw // 8)
                out_copy(kt, half).start()
                in_copy(kt + 2, half).start()

        for half in range(2):
            kt = 2 * ((iters + 1) // 2) + half
            in_copy(kt, half).wait()
            out_copy(kt - 2, half).wait()

        @pl.when(wid < F)
        def _():
            j = wid
            pltpu.sync_copy(tail_hbm.at[j], obufs.at[0, pl.ds(0, 8)])
            pltpu.sync_copy(obufs.at[0, pl.ds(0, 8)],
                            o_hbm.at[j, pl.ds(v_main // 8, 8)])

    return k(w_t, w_tail)


def _sc_embed(x_t, w, *, num_fields, batch, emb_dim):
    mesh = plsc.VectorSubcoreMesh(core_axis_name="core",
                                  subcore_axis_name="subcore")
    num_chunks = batch // _CHUNK
    num_units = num_fields * num_chunks
    units_per_worker = num_units // _NW

    @functools.partial(
        pl.kernel,
        out_type=jax.ShapeDtypeStruct((num_fields * emb_dim // 8,
                                       batch // _WIN, 8, _WIN), jnp.float32),
        mesh=mesh,
        scratch_types=[
            pltpu.VMEM((_RPW, _WIN), jnp.int32),
            pltpu.VMEM((_CHUNK, emb_dim), jnp.float32),
            pltpu.VMEM((2, 4, 8, _WIN), jnp.float32),
            pltpu.SemaphoreType.DMA,
        ],
        compiler_params=_no_layout_passes(
            pltpu.CompilerParams(use_tc_tiling_on_sc=False)),
    )
    def k(x_hbm, w_hbm, o_hbm, idx_v, rows_v, tbuf, sem):
        wid = jax.lax.axis_index("subcore") * 2 + jax.lax.axis_index("core")
        iota = jax.lax.iota(jnp.int32, 16)
        half_c = [jnp.full((16,), fp // 8, jnp.int32) for fp in range(16)]
        sub_c = [jnp.full((16,), fp % 8, jnp.int32) for fp in range(16)]
        feat_c = [jnp.full((16,), fp, jnp.int32) for fp in range(16)]

        @pl.loop(0, units_per_worker)
        def _(t):
            u = t * _NW + wid
            j = u % num_fields
            c = u // num_fields
            pltpu.sync_copy(x_hbm.at[j, pl.ds(c * _RPW, _RPW)], idx_v)
            for r in range(_RPW):
                pltpu.async_copy(
                    w_hbm.at[j].at[idx_v.at[r]],
                    rows_v.at[pl.ds(r * _WIN, _WIN)],
                    sem,
                )
            for r in range(_RPW):
                pltpu.make_async_copy(
                    w_hbm.at[j].at[idx_v.at[r]],
                    rows_v.at[pl.ds(r * _WIN, _WIN)],
                    sem,
                ).wait()

            @pl.loop(0, _CHUNK // 16)
            def _(m):
                vb = jnp.full((16,), m * 16, jnp.int32) + iota
                vtc = jnp.full((16,), m // 8, jnp.int32)
                vcc = jnp.full((16,), (m % 8) * 16, jnp.int32) + iota
                vecs = [plsc.load_gather(rows_v, [vb, feat_c[fp]])
                        for fp in range(16)]
                for fp in range(16):
                    plsc.store_scatter(
                        tbuf, [half_c[fp], vtc, sub_c[fp], vcc], vecs[fp])

            pltpu.sync_copy(tbuf, o_hbm.at[pl.ds(2 * j, 2), pl.ds(4 * c, 4)])

    return k(x_t, w)


@functools.partial(jax.jit, static_argnames=("num_fields", "batch", "emb_dim"))
def _embed_pipeline(x_t, W, *, num_fields, batch, emb_dim):
    V1 = W.shape[1]
    v_pad = ((V1 + 63) // 64) * 64
    v_main = (((V1 - 1) // _WIN) * _WIN)
    tail = jnp.pad(W[:, v_main:, :], ((0, 0), (0, v_main + 64 - V1), (0, 0)))
    w_tail = tail.reshape(num_fields, 8, 8 * emb_dim)
    w_pack = _sc_pack(jnp.transpose(W, (0, 2, 1)), w_tail, v_pad)
    w_lin = w_pack.reshape(num_fields, v_pad, emb_dim)
    return _sc_embed(x_t, w_lin, num_fields=num_fields, batch=batch,
                     emb_dim=emb_dim)


def kernel(x_cat, W):
    B, F = x_cat.shape
    _, _, ED = W.shape
    x_t = jnp.transpose(x_cat).astype(jnp.int32).reshape(F, B // _WIN, _WIN)
    out = _embed_pipeline(x_t, W, num_fields=F, batch=B, emb_dim=ED)
    return out.transpose(1, 3, 0, 2).reshape(B, F * ED)
